# baseline (speedup 1.0000x reference)
import functools

import jax
import jax.numpy as jnp
from jax import lax
from jax.experimental import pallas as pl
from jax.experimental.pallas import tpu as pltpu
from jax.experimental.pallas import tpu_sc as plsc

_TM = 512


def _vq_tc_body(x_ref, wdT_ref, b_ref, bookT_ref, z_ref, idx_ref, loss_ref):
    i = pl.program_id(0)
    k = bookT_ref.shape[1]
    z = jnp.dot(x_ref[...], wdT_ref[...], preferred_element_type=jnp.float32)
    z = z + b_ref[...]
    bookT = bookT_ref[...]
    c2 = jnp.sum(bookT * bookT, axis=0, keepdims=True)
    s = jnp.dot(z, bookT, preferred_element_type=jnp.float32)
    z2 = jnp.sum(z * z, axis=1, keepdims=True)
    d2 = (z2 - 2.0 * s) + c2
    dmin = jnp.min(d2, axis=1, keepdims=True)
    kio = lax.broadcasted_iota(jnp.int32, d2.shape, 1)
    idx = jnp.min(jnp.where(d2 == dmin, kio, k), axis=1, keepdims=True)
    z_ref[...] = z
    idx_ref[...] = idx

    @pl.when(i == 0)
    def _():
        loss_ref[...] = jnp.zeros_like(loss_ref)

    loss_ref[...] += jnp.sum(jnp.maximum(dmin, 0.0), axis=(0, 1), keepdims=True)


def _vq_tc(xf, wdT, b2, bookT):
    m, d = xf.shape
    c, k = bookT.shape
    return pl.pallas_call(
        _vq_tc_body,
        grid=(m // _TM,),
        in_specs=[
            pl.BlockSpec((_TM, d), lambda i: (i, 0)),
            pl.BlockSpec((d, c), lambda i: (0, 0)),
            pl.BlockSpec((1, c), lambda i: (0, 0)),
            pl.BlockSpec((c, k), lambda i: (0, 0)),
        ],
        out_specs=[
            pl.BlockSpec((_TM, c), lambda i: (i, 0)),
            pl.BlockSpec((_TM, 1), lambda i: (i, 0)),
            pl.BlockSpec((1, 1), lambda i: (0, 0)),
        ],
        out_shape=[
            jax.ShapeDtypeStruct((m, c), jnp.float32),
            jax.ShapeDtypeStruct((m, 1), jnp.int32),
            jax.ShapeDtypeStruct((1, 1), jnp.float32),
        ],
    )(xf, wdT, b2, bookT)


def _sc_gather(book_pad, idx_flat):
    info = plsc.get_sparse_core_info()
    nc, ns = info.num_cores, info.num_subcores
    nw = nc * ns
    b = idx_flat.shape[0]
    d = book_pad.shape[1]
    bpw = b // nw
    chunks = []
    off = 0
    while off < bpw:
        n = min(128, bpw - off)
        chunks.append((off, n))
        off += n
    mesh = plsc.VectorSubcoreMesh(core_axis_name="c", subcore_axis_name="s")

    @functools.partial(
        pl.kernel,
        out_type=jax.ShapeDtypeStruct((b, d), jnp.float32),
        mesh=mesh,
        scratch_types=[
            pltpu.VMEM((bpw,), jnp.int32),
            pltpu.VMEM((bpw, d), jnp.float32),
            pltpu.SemaphoreType.DMA,
        ],
    )
    def gk(table_hbm, idx_hbm, out_hbm, idx_v, rows_v, sem):
        wid = lax.axis_index("s") * nc + lax.axis_index("c")
        base = wid * bpw
        pltpu.sync_copy(idx_hbm.at[pl.ds(base, bpw)], idx_v)
        handles = [
            pltpu.async_copy(
                table_hbm.at[idx_v.at[pl.ds(o, n)]], rows_v.at[pl.ds(o, n)], sem
            )
            for (o, n) in chunks
        ]
        for h in handles:
            h.wait()
        pltpu.sync_copy(rows_v, out_hbm.at[pl.ds(base, bpw)])

    return gk(book_pad, idx_flat)


def kernel(x, codebook, W_down, b_down):
    b, t, dm = x.shape
    book = codebook[0]
    xf = x.reshape(b * t, dm)
    z_f, idx_f, loss_sum = _vq_tc(
        xf, W_down.T, b_down.reshape(1, -1), book.T
    )
    c = book.shape[1]
    book_pad = jnp.pad(book, ((0, 0), (0, 128 - c)))
    hard_f = _sc_gather(book_pad, idx_f.reshape(b * t))[:, :c]
    z = z_f.reshape(b, t, -1)
    code_indices = idx_f.reshape(b, t)
    hard_codes_st = hard_f.reshape(b, t, -1)
    loss = loss_sum[0, 0] / (b * t * book.shape[1])
    return (z, code_indices, hard_codes_st, loss, loss)


# speedup vs baseline: 1.4072x; 1.4072x over previous
import functools

import jax
import jax.numpy as jnp
from jax import lax
from jax.experimental import pallas as pl
from jax.experimental.pallas import tpu as pltpu
from jax.experimental.pallas import tpu_sc as plsc

_TM = 512


def _vq_tc_body(x_ref, w_ref, b_ref, book_ref, z_ref, idx_ref, loss_ref, c2h_ref):
    i = pl.program_id(0)
    book = book_ref[...]
    k = book.shape[0]

    @pl.when(i == 0)
    def _():
        c2h_ref[...] = 0.5 * jnp.sum(book * book, axis=1, keepdims=True)
        loss_ref[...] = jnp.zeros_like(loss_ref)

    z = lax.dot_general(
        x_ref[...], w_ref[...],
        dimension_numbers=(((1,), (1,)), ((), ())),
        preferred_element_type=jnp.float32,
    ) + b_ref[...]
    shatT = lax.dot_general(
        book, z,
        dimension_numbers=(((1,), (1,)), ((), ())),
        preferred_element_type=jnp.float32,
    ) - c2h_ref[...]
    smaxT = jnp.max(shatT, axis=0, keepdims=True)
    kio = lax.broadcasted_iota(jnp.int32, shatT.shape, 0)
    idxT = jnp.min(jnp.where(shatT == smaxT, kio, k), axis=0, keepdims=True)
    z_ref[...] = z
    idx_ref[...] = idxT
    part = jnp.sum(z * z, axis=(0, 1), keepdims=True) - 2.0 * jnp.sum(
        smaxT, axis=(0, 1), keepdims=True
    )
    loss_ref[...] += part


def _vq_tc(xf, w, b2, book):
    m, d = xf.shape
    k, c = book.shape
    return pl.pallas_call(
        _vq_tc_body,
        grid=(m // _TM,),
        in_specs=[
            pl.BlockSpec((_TM, d), lambda i: (i, 0)),
            pl.BlockSpec((c, d), lambda i: (0, 0)),
            pl.BlockSpec((1, c), lambda i: (0, 0)),
            pl.BlockSpec((k, c), lambda i: (0, 0)),
        ],
        out_specs=[
            pl.BlockSpec((_TM, c), lambda i: (i, 0)),
            pl.BlockSpec((1, _TM), lambda i: (0, i)),
            pl.BlockSpec((1, 1), lambda i: (0, 0)),
        ],
        out_shape=[
            jax.ShapeDtypeStruct((m, c), jnp.float32),
            jax.ShapeDtypeStruct((1, m), jnp.int32),
            jax.ShapeDtypeStruct((1, 1), jnp.float32),
        ],
        scratch_shapes=[pltpu.VMEM((k, 1), jnp.float32)],
    )(xf, w, b2, book)


def _sc_gather(book_flat, idx_flat, c):
    info = plsc.get_sparse_core_info()
    nc, ns, lanes = info.num_cores, info.num_subcores, info.num_lanes
    nw = nc * ns
    m = idx_flat.shape[0]
    bpw = m // nw
    ngrp = bpw // lanes
    mesh = plsc.VectorSubcoreMesh(core_axis_name="c", subcore_axis_name="s")

    @functools.partial(
        pl.kernel,
        out_type=jax.ShapeDtypeStruct((m * c,), jnp.float32),
        mesh=mesh,
        compiler_params=pltpu.CompilerParams(needs_layout_passes=False),
        scratch_types=[
            pltpu.VMEM((book_flat.shape[0],), jnp.float32),
            pltpu.VMEM((bpw,), jnp.int32),
            pltpu.VMEM((bpw * c,), jnp.float32),
        ],
    )
    def gk(table_hbm, idx_hbm, out_hbm, table_v, idx_v, rows_v):
        wid = lax.axis_index("s") * nc + lax.axis_index("c")
        base = wid * bpw
        pltpu.sync_copy(table_hbm, table_v)
        pltpu.sync_copy(idx_hbm.at[pl.ds(base, bpw)], idx_v)
        lane = lax.iota(jnp.int32, lanes)

        def body(g, carry):
            v16 = idx_v[pl.ds(g * lanes, lanes)]
            for j in range(lanes):
                spl = lax.gather(
                    v16,
                    jnp.full((lanes, 1), j, jnp.int32),
                    lax.GatherDimensionNumbers(
                        offset_dims=(),
                        collapsed_slice_dims=(0,),
                        start_index_map=(0,),
                    ),
                    slice_sizes=(1,),
                    mode=lax.GatherScatterMode.PROMISE_IN_BOUNDS,
                )
                flat = spl * c
                r = g * lanes + j
                for h in range(c // lanes):
                    vals = plsc.load_gather(table_v, [flat + (lane + h * lanes)])
                    rows_v[pl.ds(r * c + h * lanes, lanes)] = vals
            return carry

        lax.fori_loop(0, ngrp, body, 0)
        pltpu.sync_copy(rows_v, out_hbm.at[pl.ds(base * c, bpw * c)])

    return gk(book_flat, idx_flat)


def kernel(x, codebook, W_down, b_down):
    b, t, dm = x.shape
    book = codebook[0]
    k, c = book.shape
    xf = x.reshape(b * t, dm)
    z_f, idx_f, loss_sum = _vq_tc(xf, W_down, b_down.reshape(1, -1), book)
    hard_flat = _sc_gather(book.reshape(k * c), idx_f.reshape(b * t), c)
    z = z_f.reshape(b, t, c)
    code_indices = idx_f.reshape(b, t)
    hard_codes_st = hard_flat.reshape(b, t, c)
    loss = loss_sum[0, 0] / (b * t * c)
    return (z, code_indices, hard_codes_st, loss, loss)


# speedup vs baseline: 1.6617x; 1.1809x over previous
import functools

import jax
import jax.numpy as jnp
from jax import lax
from jax.experimental import pallas as pl
from jax.experimental.pallas import tpu as pltpu
from jax.experimental.pallas import tpu_sc as plsc

_TM = 2048
_KCHUNKS = 2


def _vq_tc_body(x_ref, w_ref, b_ref, book_ref, z_ref, idx_ref, loss_ref, aug_ref):
    i = pl.program_id(0)

    @pl.when(i == 0)
    def _():
        book = book_ref[...]
        aug_ref[...] = 0.5 * jnp.sum(book * book, axis=1, keepdims=True)
        loss_ref[...] = jnp.zeros_like(loss_ref)

    z = lax.dot_general(
        x_ref[...], w_ref[...],
        dimension_numbers=(((1,), (1,)), ((), ())),
        preferred_element_type=jnp.float32,
    ) + b_ref[...]
    kc = book_ref.shape[0] // _KCHUNKS
    ms, ids = [], []
    for j in range(_KCHUNKS):
        sj = lax.dot_general(
            book_ref[pl.ds(j * kc, kc), :], z,
            dimension_numbers=(((1,), (1,)), ((), ())),
            preferred_element_type=jnp.float32,
        ) - aug_ref[pl.ds(j * kc, kc), :]
        ms.append(jnp.max(sj, axis=0, keepdims=True))
        ids.append(
            jnp.argmax(sj, axis=0).astype(jnp.int32).reshape(1, -1) + j * kc
        )
    smaxT, idxT = ms[0], ids[0]
    for j in range(1, _KCHUNKS):
        better = smaxT >= ms[j]
        idxT = jnp.where(better, idxT, ids[j])
        smaxT = jnp.where(better, smaxT, ms[j])
    z_ref[...] = z
    idx_ref[...] = idxT
    part = jnp.sum(z * z, axis=(0, 1), keepdims=True) - 2.0 * jnp.sum(
        smaxT, axis=(0, 1), keepdims=True
    )
    loss_ref[...] += part


def _vq_tc(xf, w, b2, book):
    m, d = xf.shape
    k, c = book.shape
    return pl.pallas_call(
        _vq_tc_body,
        grid=(m // _TM,),
        in_specs=[
            pl.BlockSpec((_TM, d), lambda i: (i, 0)),
            pl.BlockSpec((c, d), lambda i: (0, 0)),
            pl.BlockSpec((1, c), lambda i: (0, 0)),
            pl.BlockSpec((k, c), lambda i: (0, 0)),
        ],
        out_specs=[
            pl.BlockSpec((_TM, c), lambda i: (i, 0)),
            pl.BlockSpec((1, _TM), lambda i: (0, i)),
            pl.BlockSpec((1, 1), lambda i: (0, 0)),
        ],
        out_shape=[
            jax.ShapeDtypeStruct((m, c), jnp.float32),
            jax.ShapeDtypeStruct((1, m), jnp.int32),
            jax.ShapeDtypeStruct((1, 1), jnp.float32),
        ],
        scratch_shapes=[pltpu.VMEM((k, 1), jnp.float32)],
    )(xf, w, b2, book)


def _sc_gather(book_flat, idx_flat, c):
    info = plsc.get_sparse_core_info()
    nc, ns, lanes = info.num_cores, info.num_subcores, info.num_lanes
    nw = nc * ns
    m = idx_flat.shape[0]
    bpw = m // nw
    ngrp = bpw // lanes
    mesh = plsc.VectorSubcoreMesh(core_axis_name="c", subcore_axis_name="s")

    @functools.partial(
        pl.kernel,
        out_type=jax.ShapeDtypeStruct((m * c,), jnp.float32),
        mesh=mesh,
        compiler_params=pltpu.CompilerParams(needs_layout_passes=False),
        scratch_types=[
            pltpu.VMEM((book_flat.shape[0],), jnp.float32),
            pltpu.VMEM((bpw,), jnp.int32),
            pltpu.VMEM((bpw * c,), jnp.float32),
        ],
    )
    def gk(table_hbm, idx_hbm, out_hbm, table_v, idx_v, rows_v):
        wid = lax.axis_index("s") * nc + lax.axis_index("c")
        base = wid * bpw
        pltpu.sync_copy(table_hbm, table_v)
        pltpu.sync_copy(idx_hbm.at[pl.ds(base, bpw)], idx_v)
        lane = lax.iota(jnp.int32, lanes)

        def body(g, carry):
            v16 = idx_v[pl.ds(g * lanes, lanes)]
            for j in range(lanes):
                spl = lax.gather(
                    v16,
                    jnp.full((lanes, 1), j, jnp.int32),
                    lax.GatherDimensionNumbers(
                        offset_dims=(),
                        collapsed_slice_dims=(0,),
                        start_index_map=(0,),
                    ),
                    slice_sizes=(1,),
                    mode=lax.GatherScatterMode.PROMISE_IN_BOUNDS,
                )
                flat = spl * c
                r = g * lanes + j
                for h in range(c // lanes):
                    vals = plsc.load_gather(table_v, [flat + (lane + h * lanes)])
                    rows_v[pl.ds(r * c + h * lanes, lanes)] = vals
            return carry

        lax.fori_loop(0, ngrp, body, 0)
        pltpu.sync_copy(rows_v, out_hbm.at[pl.ds(base * c, bpw * c)])

    return gk(book_flat, idx_flat)


def kernel(x, codebook, W_down, b_down):
    b, t, dm = x.shape
    book = codebook[0]
    k, c = book.shape
    xf = x.reshape(b * t, dm)
    z_f, idx_f, loss_sum = _vq_tc(xf, W_down, b_down.reshape(1, -1), book)
    hard_flat = _sc_gather(book.reshape(k * c), idx_f.reshape(b * t), c)
    z = z_f.reshape(b, t, c)
    code_indices = idx_f.reshape(b, t)
    hard_codes_st = hard_flat.reshape(b, t, c)
    loss = loss_sum[0, 0] / (b * t * c)
    return (z, code_indices, hard_codes_st, loss, loss)
